# baseline (speedup 1.0000x reference)
import functools

import jax
import jax.numpy as jnp
from jax import lax
from jax.experimental import pallas as pl
from jax.experimental.pallas import tpu as pltpu
from jax.experimental.pallas import tpu_sc as plsc

F32 = jnp.float32

NC = 2
NS = 16
NW = NC * NS
CH = 128


def _round_up(a, m):
    return (a + m - 1) // m * m


def _tables_body(x_ref, wa_ref, wb_ref, b1_ref, ta_ref, tb_ref):
    xb = x_ref[...]
    ta_ref[...] = jnp.dot(xb, wa_ref[...], preferred_element_type=F32)
    tb_ref[...] = jnp.dot(xb, wb_ref[...],
                          preferred_element_type=F32) + b1_ref[...]


def _build_tables(x, waT, wbT, b1):
    n, d = x.shape
    bn = 1000
    grid = n // bn
    return pl.pallas_call(
        _tables_body,
        grid=(grid,),
        in_specs=[
            pl.BlockSpec((bn, d), lambda i: (i, 0)),
            pl.BlockSpec((d, d), lambda i: (0, 0)),
            pl.BlockSpec((d, d), lambda i: (0, 0)),
            pl.BlockSpec((1, d), lambda i: (0, 0)),
        ],
        out_specs=[
            pl.BlockSpec((bn, d), lambda i: (i, 0)),
            pl.BlockSpec((bn, d), lambda i: (i, 0)),
        ],
        out_shape=[
            jax.ShapeDtypeStruct((n, d), F32),
            jax.ShapeDtypeStruct((n, d), F32),
        ],
    )(x, waT, wbT, b1)


def _gather_kernel(e_pad, dw, tiled):
    epw = e_pad // NW
    nchunk = epw // CH
    npair = nchunk // 2
    nslice = dw // 16
    mesh = plsc.VectorSubcoreMesh(
        core_axis_name="c", subcore_axis_name="s",
        num_cores=NC, num_subcores=NS)
    params = None if tiled else pltpu.CompilerParams(use_tc_tiling_on_sc=False)

    @functools.partial(
        pl.kernel,
        out_type=jax.ShapeDtypeStruct((e_pad, dw), F32),
        mesh=mesh,
        compiler_params=params,
        scratch_types=[
            pltpu.VMEM((CH,), jnp.int32),
            pltpu.VMEM((CH,), jnp.int32),
            pltpu.VMEM((CH, dw), F32),
            pltpu.VMEM((CH, dw), F32),
            pltpu.VMEM((CH,), jnp.int32),
            pltpu.VMEM((CH,), jnp.int32),
            pltpu.VMEM((CH, dw), F32),
            pltpu.VMEM((CH, dw), F32),
            pltpu.SemaphoreType.DMA,
            pltpu.SemaphoreType.DMA,
            pltpu.SemaphoreType.DMA,
            pltpu.SemaphoreType.DMA,
        ],
    )
    def k(ta_hbm, tb_hbm, send_hbm, rec_hbm, h_hbm,
          sidx0, ridx0, bs0, br0, sidx1, ridx1, bs1, br1, g0, g1, s0, s1):
        wid = lax.axis_index("s") * NC + lax.axis_index("c")
        base = wid * epw

        def addstore(bs, br, off, sem):
            def addrow(r, c2):
                for j in range(nslice):
                    sl = pl.ds(j * 16, 16)
                    bs[r, sl] = bs[r, sl] + br[r, sl]
                return c2

            lax.fori_loop(0, CH, addrow, 0)
            pltpu.async_copy(bs, h_hbm.at[pl.ds(off, CH)], sem)

        def pair(j, carry):
            off0 = base + (2 * j) * CH
            off1 = off0 + CH

            @pl.when(j > 0)
            def _drain():
                pltpu.make_async_copy(
                    bs0, h_hbm.at[pl.ds(base, CH)], s0).wait()
                pltpu.make_async_copy(
                    bs1, h_hbm.at[pl.ds(base, CH)], s1).wait()

            pltpu.sync_copy(send_hbm.at[pl.ds(off0, CH)], sidx0)
            pltpu.sync_copy(rec_hbm.at[pl.ds(off0, CH)], ridx0)
            pltpu.async_copy(ta_hbm.at[sidx0], bs0, g0)
            pltpu.async_copy(tb_hbm.at[ridx0], br0, g0)
            pltpu.sync_copy(send_hbm.at[pl.ds(off1, CH)], sidx1)
            pltpu.sync_copy(rec_hbm.at[pl.ds(off1, CH)], ridx1)
            pltpu.async_copy(ta_hbm.at[sidx1], bs1, g1)
            pltpu.async_copy(tb_hbm.at[ridx1], br1, g1)

            pltpu.make_async_copy(ta_hbm.at[sidx0], bs0, g0).wait()
            pltpu.make_async_copy(tb_hbm.at[ridx0], br0, g0).wait()
            addstore(bs0, br0, off0, s0)
            pltpu.make_async_copy(ta_hbm.at[sidx1], bs1, g1).wait()
            pltpu.make_async_copy(tb_hbm.at[ridx1], br1, g1).wait()
            addstore(bs1, br1, off1, s1)
            return carry

        lax.fori_loop(0, npair, pair, 0)
        pltpu.make_async_copy(bs0, h_hbm.at[pl.ds(base, CH)], s0).wait()
        pltpu.make_async_copy(bs1, h_hbm.at[pl.ds(base, CH)], s1).wait()

    return k


def _edge_body(h_ref, pd_ref, w1c_ref, w2_ref, b2_ref, m_ref):
    d = w2_ref.shape[0]
    be = h_ref.shape[0]
    h1 = h_ref[...]
    pd = pd_ref[...]
    dz = pd.reshape(be // 8, 8, 16)
    d2 = jnp.sum(dz * dz, axis=2)
    dist = jnp.sqrt(d2 + 1e-12).reshape(be, 1)
    h = jax.nn.silu(h1 + dist * w1c_ref[...])
    t = jnp.dot(h, w2_ref[...], preferred_element_type=F32) + b2_ref[...]
    m_ref[...] = jax.nn.silu(t)


def _edge_mlp(h, pd2, w1c, w2T, b2):
    e_pad, d = h.shape
    be = 512
    grid = e_pad // be
    return pl.pallas_call(
        _edge_body,
        grid=(grid,),
        in_specs=[
            pl.BlockSpec((be, d), lambda i: (i, 0)),
            pl.BlockSpec((be // 8, d), lambda i: (i, 0)),
            pl.BlockSpec((1, d), lambda i: (0, 0)),
            pl.BlockSpec((d, d), lambda i: (0, 0)),
            pl.BlockSpec((1, d), lambda i: (0, 0)),
        ],
        out_specs=pl.BlockSpec((be, d), lambda i: (i, 0)),
        out_shape=jax.ShapeDtypeStruct((e_pad, d), F32),
    )(h, pd2, w1c, w2T, b2)


def _agg_kernel(e_pad, n_sh, d):
    epw = e_pad // NW
    nchunk = epw // CH
    rows_per_tile = n_sh // NS
    ozchunk = rows_per_tile // CH
    mesh = plsc.VectorSubcoreMesh(
        core_axis_name="c", subcore_axis_name="s",
        num_cores=NC, num_subcores=NS)

    @functools.partial(
        pl.kernel,
        out_type=jax.ShapeDtypeStruct((NC, n_sh, d), F32),
        mesh=mesh,
        scratch_types=[
            pltpu.VMEM((CH,), jnp.int32),
            pltpu.VMEM((CH, d), F32),
            pltpu.VMEM_SHARED((n_sh, d), F32),
        ],
    )
    def k(rec_hbm, m_hbm, out_hbm, ridx, mbuf, shared):
        cid = lax.axis_index("c")
        sid = lax.axis_index("s")
        wid = sid * NC + cid
        tbase = sid * rows_per_tile

        def zrow(r, c2):
            for j in range(d // 16):
                mbuf[r, pl.ds(j * 16, 16)] = jnp.zeros((16,), F32)
            return c2

        lax.fori_loop(0, CH, zrow, 0)

        def zchunk(i, c2):
            pltpu.sync_copy(mbuf, shared.at[pl.ds(tbase + i * CH, CH)])
            return c2

        lax.fori_loop(0, ozchunk, zchunk, 0)
        plsc.subcore_barrier()

        base = wid * epw

        def chunk(i, c2):
            off = base + i * CH
            pltpu.sync_copy(rec_hbm.at[pl.ds(off, CH)], ridx)
            pltpu.sync_copy(m_hbm.at[pl.ds(off, CH)], mbuf)
            pltpu.sync_copy(mbuf, shared.at[ridx], add=True)
            return c2

        lax.fori_loop(0, nchunk, chunk, 0)
        plsc.subcore_barrier()

        def ochunk(i, c2):
            sl = pl.ds(tbase + i * CH, CH)
            pltpu.sync_copy(shared.at[sl], out_hbm.at[cid, sl])
            return c2

        lax.fori_loop(0, ozchunk, ochunk, 0)

    return k


def _node_body(x_ref, p0_ref, p1_ref, w3a_ref, w3b_ref, b3_ref,
               w4_ref, b4_ref, out_ref):
    xb = x_ref[...]
    aggr = p0_ref[...] + p1_ref[...]
    u = jax.nn.silu(
        jnp.dot(xb, w3a_ref[...], preferred_element_type=F32)
        + jnp.dot(aggr, w3b_ref[...], preferred_element_type=F32)
        + b3_ref[...])
    out_ref[...] = jnp.dot(u, w4_ref[...], preferred_element_type=F32) \
        + b4_ref[...]


def _node_mlp(x, p0, p1, w3aT, w3bT, b3, w4T, b4):
    n, d = x.shape
    bn = 1000
    grid = n // bn
    return pl.pallas_call(
        _node_body,
        grid=(grid,),
        in_specs=[
            pl.BlockSpec((bn, d), lambda i: (i, 0)),
            pl.BlockSpec((bn, d), lambda i: (i, 0)),
            pl.BlockSpec((bn, d), lambda i: (i, 0)),
            pl.BlockSpec((d, d), lambda i: (0, 0)),
            pl.BlockSpec((d, d), lambda i: (0, 0)),
            pl.BlockSpec((1, d), lambda i: (0, 0)),
            pl.BlockSpec((d, d), lambda i: (0, 0)),
            pl.BlockSpec((1, d), lambda i: (0, 0)),
        ],
        out_specs=pl.BlockSpec((bn, d), lambda i: (i, 0)),
        out_shape=jax.ShapeDtypeStruct((n, d), F32),
    )(x, p0, p1, w3aT, w3bT, b3, w4T, b4)


def kernel(x, pos, edge_index, W1, b1, W2, b2, W3, b3, W4, b4):
    n, d = x.shape
    e = edge_index.shape[1]
    e_pad = _round_up(e, NW * CH * 2)
    n_sh = _round_up(n + 1, NS * CH)

    waT = W1[:, :d].T
    wbT = W1[:, d:2 * d].T
    w1c = W1[:, 2 * d].reshape(1, d)
    b1r = b1.reshape(1, d)
    w2T = W2.T
    b2r = b2.reshape(1, d)
    w3aT = W3[:, :d].T
    w3bT = W3[:, d:].T
    b3r = b3.reshape(1, d)
    w4T = W4.T
    b4r = b4.reshape(1, d)

    send = edge_index[0]
    rec = edge_index[1]
    pad = e_pad - e
    send_p = jnp.concatenate([send, jnp.zeros((pad,), jnp.int32)])
    rec_p = jnp.concatenate([rec, jnp.zeros((pad,), jnp.int32)])
    rec_agg = jnp.concatenate([rec, jnp.full((pad,), n, jnp.int32)])

    pp = jnp.concatenate([pos, jnp.zeros((n, 13), F32)], axis=1)

    ta, tb = _build_tables(x, waT, wbT, b1r)
    h1 = _gather_kernel(e_pad, d, True)(ta, tb, send_p, rec_p)
    pd = _gather_kernel(e_pad, 16, False)(pp, -pp, send_p, rec_p)
    pd2 = pd.reshape(e_pad // 8, d)
    msgs = _edge_mlp(h1, pd2, w1c, w2T, b2r)
    partials = _agg_kernel(e_pad, n_sh, d)(rec_agg, msgs)
    update = _node_mlp(x, partials[0, :n], partials[1, :n],
                       w3aT, w3bT, b3r, w4T, b4r)
    return update


# speedup vs baseline: 1.1335x; 1.1335x over previous
import functools

import jax
import jax.numpy as jnp
from jax import lax
from jax.experimental import pallas as pl
from jax.experimental.pallas import tpu as pltpu
from jax.experimental.pallas import tpu_sc as plsc

F32 = jnp.float32

NC = 2
NS = 16
NW = NC * NS
CH = 128


def _round_up(a, m):
    return (a + m - 1) // m * m


def _tables_body(x_ref, wa_ref, wb_ref, b1_ref, ta_ref, tb_ref):
    xb = x_ref[...]
    ta_ref[...] = jnp.dot(xb, wa_ref[...], preferred_element_type=F32)
    tb_ref[...] = jnp.dot(xb, wb_ref[...],
                          preferred_element_type=F32) + b1_ref[...]


def _build_tables(x, waT, wbT, b1):
    n, d = x.shape
    bn = 1000
    grid = n // bn
    return pl.pallas_call(
        _tables_body,
        grid=(grid,),
        in_specs=[
            pl.BlockSpec((bn, d), lambda i: (i, 0)),
            pl.BlockSpec((d, d), lambda i: (0, 0)),
            pl.BlockSpec((d, d), lambda i: (0, 0)),
            pl.BlockSpec((1, d), lambda i: (0, 0)),
        ],
        out_specs=[
            pl.BlockSpec((bn, d), lambda i: (i, 0)),
            pl.BlockSpec((bn, d), lambda i: (i, 0)),
        ],
        out_shape=[
            jax.ShapeDtypeStruct((n, d), F32),
            jax.ShapeDtypeStruct((n, d), F32),
        ],
    )(x, waT, wbT, b1)


def _gather_kernel(e_pad, dw, tiled, p0, p1):
    assert NS * (p0 + p1) * 2 * CH == e_pad
    nslice = dw // 16
    mesh = plsc.VectorSubcoreMesh(
        core_axis_name="c", subcore_axis_name="s",
        num_cores=NC, num_subcores=NS)
    params = None if tiled else pltpu.CompilerParams(use_tc_tiling_on_sc=False)

    @functools.partial(
        pl.kernel,
        out_type=jax.ShapeDtypeStruct((e_pad, dw), F32),
        mesh=mesh,
        compiler_params=params,
        scratch_types=[
            pltpu.VMEM((CH,), jnp.int32),
            pltpu.VMEM((CH,), jnp.int32),
            pltpu.VMEM((CH, dw), F32),
            pltpu.VMEM((CH, dw), F32),
            pltpu.VMEM((CH,), jnp.int32),
            pltpu.VMEM((CH,), jnp.int32),
            pltpu.VMEM((CH, dw), F32),
            pltpu.VMEM((CH, dw), F32),
            pltpu.SemaphoreType.DMA,
            pltpu.SemaphoreType.DMA,
            pltpu.SemaphoreType.DMA,
            pltpu.SemaphoreType.DMA,
        ],
    )
    def k(ta_hbm, tb_hbm, send_hbm, rec_hbm, h_hbm,
          sidx0, ridx0, bs0, br0, sidx1, ridx1, bs1, br1, g0, g1, s0, s1):
        cid = lax.axis_index("c")
        sid = lax.axis_index("s")
        npair = jnp.where(cid == 0, p0, p1)
        base = jnp.where(cid == 0, sid * p0, NS * p0 + sid * p1) * 2 * CH

        def addstore(bs, br, off, sem):
            def addrow(r, c2):
                for j in range(nslice):
                    sl = pl.ds(j * 16, 16)
                    bs[r, sl] = bs[r, sl] + br[r, sl]
                return c2

            lax.fori_loop(0, CH, addrow, 0)
            pltpu.async_copy(bs, h_hbm.at[pl.ds(off, CH)], sem)

        def pair(j, carry):
            off0 = base + (2 * j) * CH
            off1 = off0 + CH

            @pl.when(j > 0)
            def _drain():
                pltpu.make_async_copy(
                    bs0, h_hbm.at[pl.ds(base, CH)], s0).wait()
                pltpu.make_async_copy(
                    bs1, h_hbm.at[pl.ds(base, CH)], s1).wait()

            pltpu.sync_copy(send_hbm.at[pl.ds(off0, CH)], sidx0)
            pltpu.sync_copy(rec_hbm.at[pl.ds(off0, CH)], ridx0)
            pltpu.async_copy(ta_hbm.at[sidx0], bs0, g0)
            pltpu.async_copy(tb_hbm.at[ridx0], br0, g0)
            pltpu.sync_copy(send_hbm.at[pl.ds(off1, CH)], sidx1)
            pltpu.sync_copy(rec_hbm.at[pl.ds(off1, CH)], ridx1)
            pltpu.async_copy(ta_hbm.at[sidx1], bs1, g1)
            pltpu.async_copy(tb_hbm.at[ridx1], br1, g1)

            pltpu.make_async_copy(ta_hbm.at[sidx0], bs0, g0).wait()
            pltpu.make_async_copy(tb_hbm.at[ridx0], br0, g0).wait()
            addstore(bs0, br0, off0, s0)
            pltpu.make_async_copy(ta_hbm.at[sidx1], bs1, g1).wait()
            pltpu.make_async_copy(tb_hbm.at[ridx1], br1, g1).wait()
            addstore(bs1, br1, off1, s1)
            return carry

        lax.fori_loop(0, npair, pair, 0)
        pltpu.make_async_copy(bs0, h_hbm.at[pl.ds(base, CH)], s0).wait()
        pltpu.make_async_copy(bs1, h_hbm.at[pl.ds(base, CH)], s1).wait()

    return k


def _edge_body(h_ref, pd_ref, w1c_ref, w2_ref, b2_ref, m_ref):
    d = w2_ref.shape[0]
    be = h_ref.shape[0]
    h1 = h_ref[...]
    pd = pd_ref[...]
    dz = pd.reshape(be // 8, 8, 16)
    d2 = jnp.sum(dz * dz, axis=2)
    dist = jnp.sqrt(d2 + 1e-12).reshape(be, 1)
    h = jax.nn.silu(h1 + dist * w1c_ref[...])
    t = jnp.dot(h, w2_ref[...], preferred_element_type=F32) + b2_ref[...]
    m_ref[...] = jax.nn.silu(t)


def _edge_mlp(h, pd2, w1c, w2T, b2):
    e_pad, d = h.shape
    be = 1024
    grid = e_pad // be
    return pl.pallas_call(
        _edge_body,
        grid=(grid,),
        in_specs=[
            pl.BlockSpec((be, d), lambda i: (i, 0)),
            pl.BlockSpec((be // 8, d), lambda i: (i, 0)),
            pl.BlockSpec((1, d), lambda i: (0, 0)),
            pl.BlockSpec((d, d), lambda i: (0, 0)),
            pl.BlockSpec((1, d), lambda i: (0, 0)),
        ],
        out_specs=pl.BlockSpec((be, d), lambda i: (i, 0)),
        out_shape=jax.ShapeDtypeStruct((e_pad, d), F32),
    )(h, pd2, w1c, w2T, b2)


def _agg_kernel(e_pad, n_sh, d):
    epw = e_pad // NW
    nchunk = epw // CH
    rows_per_tile = n_sh // NS
    ozchunk = rows_per_tile // CH
    mesh = plsc.VectorSubcoreMesh(
        core_axis_name="c", subcore_axis_name="s",
        num_cores=NC, num_subcores=NS)

    @functools.partial(
        pl.kernel,
        out_type=jax.ShapeDtypeStruct((NC, n_sh, d), F32),
        mesh=mesh,
        scratch_types=[
            pltpu.VMEM((CH,), jnp.int32),
            pltpu.VMEM((CH, d), F32),
            pltpu.VMEM_SHARED((n_sh, d), F32),
        ],
    )
    def k(rec_hbm, m_hbm, out_hbm, ridx, mbuf, shared):
        cid = lax.axis_index("c")
        sid = lax.axis_index("s")
        wid = sid * NC + cid
        tbase = sid * rows_per_tile

        def zrow(r, c2):
            for j in range(d // 16):
                mbuf[r, pl.ds(j * 16, 16)] = jnp.zeros((16,), F32)
            return c2

        lax.fori_loop(0, CH, zrow, 0)

        def zchunk(i, c2):
            pltpu.sync_copy(mbuf, shared.at[pl.ds(tbase + i * CH, CH)])
            return c2

        lax.fori_loop(0, ozchunk, zchunk, 0)
        plsc.subcore_barrier()

        base = wid * epw

        def chunk(i, c2):
            off = base + i * CH
            pltpu.sync_copy(rec_hbm.at[pl.ds(off, CH)], ridx)
            pltpu.sync_copy(m_hbm.at[pl.ds(off, CH)], mbuf)
            pltpu.sync_copy(mbuf, shared.at[ridx], add=True)
            return c2

        lax.fori_loop(0, nchunk, chunk, 0)
        plsc.subcore_barrier()

        def ochunk(i, c2):
            sl = pl.ds(tbase + i * CH, CH)
            pltpu.sync_copy(shared.at[sl], out_hbm.at[cid, sl])
            return c2

        lax.fori_loop(0, ozchunk, ochunk, 0)

    return k


def _node_body(x_ref, p0_ref, p1_ref, w3a_ref, w3b_ref, b3_ref,
               w4_ref, b4_ref, out_ref):
    xb = x_ref[...]
    aggr = p0_ref[...] + p1_ref[...]
    u = jax.nn.silu(
        jnp.dot(xb, w3a_ref[...], preferred_element_type=F32)
        + jnp.dot(aggr, w3b_ref[...], preferred_element_type=F32)
        + b3_ref[...])
    out_ref[...] = jnp.dot(u, w4_ref[...], preferred_element_type=F32) \
        + b4_ref[...]


def _node_mlp(x, p0, p1, w3aT, w3bT, b3, w4T, b4):
    n, d = x.shape
    bn = 1000
    grid = n // bn
    return pl.pallas_call(
        _node_body,
        grid=(grid,),
        in_specs=[
            pl.BlockSpec((bn, d), lambda i: (i, 0)),
            pl.BlockSpec((bn, d), lambda i: (i, 0)),
            pl.BlockSpec((bn, d), lambda i: (i, 0)),
            pl.BlockSpec((d, d), lambda i: (0, 0)),
            pl.BlockSpec((d, d), lambda i: (0, 0)),
            pl.BlockSpec((1, d), lambda i: (0, 0)),
            pl.BlockSpec((d, d), lambda i: (0, 0)),
            pl.BlockSpec((1, d), lambda i: (0, 0)),
        ],
        out_specs=pl.BlockSpec((bn, d), lambda i: (i, 0)),
        out_shape=jax.ShapeDtypeStruct((n, d), F32),
    )(x, p0, p1, w3aT, w3bT, b3, w4T, b4)


def kernel(x, pos, edge_index, W1, b1, W2, b2, W3, b3, W4, b4):
    n, d = x.shape
    e = edge_index.shape[1]
    e_pad = _round_up(e, NW * CH * 2)
    n_sh = _round_up(n + 1, NS * CH)

    waT = W1[:, :d].T
    wbT = W1[:, d:2 * d].T
    w1c = W1[:, 2 * d].reshape(1, d)
    b1r = b1.reshape(1, d)
    w2T = W2.T
    b2r = b2.reshape(1, d)
    w3aT = W3[:, :d].T
    w3bT = W3[:, d:].T
    b3r = b3.reshape(1, d)
    w4T = W4.T
    b4r = b4.reshape(1, d)

    send = edge_index[0]
    rec = edge_index[1]
    pad = e_pad - e
    send_p = jnp.concatenate([send, jnp.zeros((pad,), jnp.int32)])
    rec_p = jnp.concatenate([rec, jnp.zeros((pad,), jnp.int32)])
    rec_agg = jnp.concatenate([rec, jnp.full((pad,), n, jnp.int32)])

    pp = jnp.concatenate([pos, jnp.zeros((n, 13), F32)], axis=1)

    ta, tb = _build_tables(x, waT, wbT, b1r)
    h1 = _gather_kernel(e_pad, d, True, 55, 25)(ta, tb, send_p, rec_p)
    pd = _gather_kernel(e_pad, 16, False, 43, 37)(pp, -pp, send_p, rec_p)
    pd2 = pd.reshape(e_pad // 8, d)
    msgs = _edge_mlp(h1, pd2, w1c, w2T, b2r)
    partials = _agg_kernel(e_pad, n_sh, d)(rec_agg, msgs)
    update = _node_mlp(x, partials[0, :n], partials[1, :n],
                       w3aT, w3bT, b3r, w4T, b4r)
    return update


# speedup vs baseline: 1.5139x; 1.3356x over previous
import functools

import jax
import jax.numpy as jnp
from jax import lax
from jax.experimental import pallas as pl
from jax.experimental.pallas import tpu as pltpu
from jax.experimental.pallas import tpu_sc as plsc

F32 = jnp.float32

NC = 2
NS = 16
NW = NC * NS
CH = 128


def _round_up(a, m):
    return (a + m - 1) // m * m


def _tables_body(x_ref, wa_ref, wb_ref, b1_ref, ta_ref, tb_ref):
    xb = x_ref[...]
    ta_ref[...] = jnp.dot(xb, wa_ref[...], preferred_element_type=F32)
    tb_ref[...] = jnp.dot(xb, wb_ref[...],
                          preferred_element_type=F32) + b1_ref[...]


def _build_tables(x, waT, wbT, b1):
    n, d = x.shape
    bn = 1000
    grid = n // bn
    return pl.pallas_call(
        _tables_body,
        grid=(grid,),
        in_specs=[
            pl.BlockSpec((bn, d), lambda i: (i, 0)),
            pl.BlockSpec((d, d), lambda i: (0, 0)),
            pl.BlockSpec((d, d), lambda i: (0, 0)),
            pl.BlockSpec((1, d), lambda i: (0, 0)),
        ],
        out_specs=[
            pl.BlockSpec((bn, d), lambda i: (i, 0)),
            pl.BlockSpec((bn, d), lambda i: (i, 0)),
        ],
        out_shape=[
            jax.ShapeDtypeStruct((n, d), F32),
            jax.ShapeDtypeStruct((n, d), F32),
        ],
    )(x, waT, wbT, b1)


def _gather_kernel(e_pad, dw, tiled, p0, p1):
    assert NS * (p0 + p1) * 2 * CH == e_pad
    nslice = dw // 16
    mesh = plsc.VectorSubcoreMesh(
        core_axis_name="c", subcore_axis_name="s",
        num_cores=NC, num_subcores=NS)
    params = None if tiled else pltpu.CompilerParams(use_tc_tiling_on_sc=False)

    @functools.partial(
        pl.kernel,
        out_type=jax.ShapeDtypeStruct((e_pad, dw), F32),
        mesh=mesh,
        compiler_params=params,
        scratch_types=[
            pltpu.VMEM((CH,), jnp.int32),
            pltpu.VMEM((CH,), jnp.int32),
            pltpu.VMEM((CH, dw), F32),
            pltpu.VMEM((CH, dw), F32),
            pltpu.VMEM((CH,), jnp.int32),
            pltpu.VMEM((CH,), jnp.int32),
            pltpu.VMEM((CH, dw), F32),
            pltpu.VMEM((CH, dw), F32),
            pltpu.SemaphoreType.DMA,
            pltpu.SemaphoreType.DMA,
            pltpu.SemaphoreType.DMA,
            pltpu.SemaphoreType.DMA,
        ],
    )
    def k(ta_hbm, tb_hbm, send_hbm, rec_hbm, h_hbm,
          sidx0, ridx0, bs0, br0, sidx1, ridx1, bs1, br1, g0, g1, s0, s1):
        cid = lax.axis_index("c")
        sid = lax.axis_index("s")
        npair = jnp.where(cid == 0, p0, p1)
        base = jnp.where(cid == 0, sid * p0, NS * p0 + sid * p1) * 2 * CH

        def addstore(bs, br, off, sem):
            def addrow(r, c2):
                for j in range(nslice):
                    sl = pl.ds(j * 16, 16)
                    bs[r, sl] = bs[r, sl] + br[r, sl]
                return c2

            lax.fori_loop(0, CH, addrow, 0)
            pltpu.async_copy(bs, h_hbm.at[pl.ds(off, CH)], sem)

        def pair(j, carry):
            off0 = base + (2 * j) * CH
            off1 = off0 + CH

            @pl.when(j > 0)
            def _drain():
                pltpu.make_async_copy(
                    bs0, h_hbm.at[pl.ds(base, CH)], s0).wait()
                pltpu.make_async_copy(
                    bs1, h_hbm.at[pl.ds(base, CH)], s1).wait()

            pltpu.sync_copy(send_hbm.at[pl.ds(off0, CH)], sidx0)
            pltpu.sync_copy(rec_hbm.at[pl.ds(off0, CH)], ridx0)
            pltpu.async_copy(ta_hbm.at[sidx0], bs0, g0)
            pltpu.async_copy(tb_hbm.at[ridx0], br0, g0)
            pltpu.sync_copy(send_hbm.at[pl.ds(off1, CH)], sidx1)
            pltpu.sync_copy(rec_hbm.at[pl.ds(off1, CH)], ridx1)
            pltpu.async_copy(ta_hbm.at[sidx1], bs1, g1)
            pltpu.async_copy(tb_hbm.at[ridx1], br1, g1)

            pltpu.make_async_copy(ta_hbm.at[sidx0], bs0, g0).wait()
            pltpu.make_async_copy(tb_hbm.at[ridx0], br0, g0).wait()
            addstore(bs0, br0, off0, s0)
            pltpu.make_async_copy(ta_hbm.at[sidx1], bs1, g1).wait()
            pltpu.make_async_copy(tb_hbm.at[ridx1], br1, g1).wait()
            addstore(bs1, br1, off1, s1)
            return carry

        lax.fori_loop(0, npair, pair, 0)
        pltpu.make_async_copy(bs0, h_hbm.at[pl.ds(base, CH)], s0).wait()
        pltpu.make_async_copy(bs1, h_hbm.at[pl.ds(base, CH)], s1).wait()

    return k


def _edge_body(h_ref, pd_ref, w1c_ref, w2_ref, b2_ref, m_ref):
    d = w2_ref.shape[0]
    be = h_ref.shape[0]
    h1 = h_ref[...]
    pd = pd_ref[...]
    dz = pd.reshape(be // 8, 8, 16)
    d2 = jnp.sum(dz * dz, axis=2)
    dist = jnp.sqrt(d2 + 1e-12).reshape(be, 1)
    h = jax.nn.silu(h1 + dist * w1c_ref[...])
    t = jnp.dot(h, w2_ref[...], preferred_element_type=F32) + b2_ref[...]
    m_ref[...] = jax.nn.silu(t)


def _edge_mlp(h, pd2, w1c, w2T, b2):
    e_pad, d = h.shape
    be = 1024
    grid = e_pad // be
    return pl.pallas_call(
        _edge_body,
        grid=(grid,),
        in_specs=[
            pl.BlockSpec((be, d), lambda i: (i, 0)),
            pl.BlockSpec((be // 8, d), lambda i: (i, 0)),
            pl.BlockSpec((1, d), lambda i: (0, 0)),
            pl.BlockSpec((d, d), lambda i: (0, 0)),
            pl.BlockSpec((1, d), lambda i: (0, 0)),
        ],
        out_specs=pl.BlockSpec((be, d), lambda i: (i, 0)),
        out_shape=jax.ShapeDtypeStruct((e_pad, d), F32),
    )(h, pd2, w1c, w2T, b2)


def _agg_kernel(e_pad, n_sh, d):
    epw = e_pad // NW
    nchunk = epw // CH
    rows_per_tile = n_sh // NS
    ozchunk = rows_per_tile // CH
    mesh = plsc.VectorSubcoreMesh(
        core_axis_name="c", subcore_axis_name="s",
        num_cores=NC, num_subcores=NS)

    @functools.partial(
        pl.kernel,
        out_type=jax.ShapeDtypeStruct((NC, n_sh, d), F32),
        mesh=mesh,
        scratch_types=[
            pltpu.VMEM((CH,), jnp.int32),
            pltpu.VMEM((CH, d), F32),
            pltpu.VMEM_SHARED((n_sh, d), F32),
        ],
    )
    def k(rec_hbm, m_hbm, out_hbm, ridx, mbuf, shared):
        cid = lax.axis_index("c")
        sid = lax.axis_index("s")
        wid = sid * NC + cid
        tbase = sid * rows_per_tile

        def zrow(r, c2):
            for j in range(d // 16):
                mbuf[r, pl.ds(j * 16, 16)] = jnp.zeros((16,), F32)
            return c2

        lax.fori_loop(0, CH, zrow, 0)

        def zchunk(i, c2):
            pltpu.sync_copy(mbuf, shared.at[pl.ds(tbase + i * CH, CH)])
            return c2

        lax.fori_loop(0, ozchunk, zchunk, 0)
        plsc.subcore_barrier()

        base = wid * epw

        def chunk(i, c2):
            off = base + i * CH
            pltpu.sync_copy(rec_hbm.at[pl.ds(off, CH)], ridx)
            pltpu.sync_copy(m_hbm.at[pl.ds(off, CH)], mbuf)
            pltpu.sync_copy(mbuf, shared.at[ridx], add=True)
            return c2

        lax.fori_loop(0, nchunk, chunk, 0)
        plsc.subcore_barrier()

        def ochunk(i, c2):
            sl = pl.ds(tbase + i * CH, CH)
            pltpu.sync_copy(shared.at[sl], out_hbm.at[cid, sl])
            return c2

        lax.fori_loop(0, ozchunk, ochunk, 0)

    return k


def _node_body(x_ref, p0_ref, p1_ref, w3a_ref, w3b_ref, b3_ref,
               w4_ref, b4_ref, out_ref):
    xb = x_ref[...]
    aggr = p0_ref[...] + p1_ref[...]
    u = jax.nn.silu(
        jnp.dot(xb, w3a_ref[...], preferred_element_type=F32)
        + jnp.dot(aggr, w3b_ref[...], preferred_element_type=F32)
        + b3_ref[...])
    out_ref[...] = jnp.dot(u, w4_ref[...], preferred_element_type=F32) \
        + b4_ref[...]


def _node_mlp(x, p0, p1, w3aT, w3bT, b3, w4T, b4):
    n, d = x.shape
    bn = 1000
    grid = n // bn
    return pl.pallas_call(
        _node_body,
        grid=(grid,),
        in_specs=[
            pl.BlockSpec((bn, d), lambda i: (i, 0)),
            pl.BlockSpec((bn, d), lambda i: (i, 0)),
            pl.BlockSpec((bn, d), lambda i: (i, 0)),
            pl.BlockSpec((d, d), lambda i: (0, 0)),
            pl.BlockSpec((d, d), lambda i: (0, 0)),
            pl.BlockSpec((1, d), lambda i: (0, 0)),
            pl.BlockSpec((d, d), lambda i: (0, 0)),
            pl.BlockSpec((1, d), lambda i: (0, 0)),
        ],
        out_specs=pl.BlockSpec((bn, d), lambda i: (i, 0)),
        out_shape=jax.ShapeDtypeStruct((n, d), F32),
    )(x, p0, p1, w3aT, w3bT, b3, w4T, b4)


def kernel(x, pos, edge_index, W1, b1, W2, b2, W3, b3, W4, b4):
    n, d = x.shape
    e = edge_index.shape[1]
    e_pad = _round_up(e, NW * CH * 2)
    n_sh = _round_up(n + 1, NS * CH)

    waT = W1[:, :d].T
    wbT = W1[:, d:2 * d].T
    w1c = W1[:, 2 * d].reshape(1, d)
    b1r = b1.reshape(1, d)
    w2T = W2.T
    b2r = b2.reshape(1, d)
    w3aT = W3[:, :d].T
    w3bT = W3[:, d:].T
    b3r = b3.reshape(1, d)
    w4T = W4.T
    b4r = b4.reshape(1, d)

    send = edge_index[0]
    rec = edge_index[1]
    pad = e_pad - e
    spread = (jnp.arange(pad, dtype=jnp.int32) * 37) % n
    send_p = jnp.concatenate([send, spread])
    rec_p = jnp.concatenate([rec, spread])
    rec_agg = jnp.concatenate([rec, jnp.full((pad,), n, jnp.int32)])

    pp = jnp.concatenate([pos, jnp.zeros((n, 13), F32)], axis=1)

    ta, tb = _build_tables(x, waT, wbT, b1r)
    h1 = _gather_kernel(e_pad, d, True, 55, 25)(ta, tb, send_p, rec_p)
    pd = _gather_kernel(e_pad, 16, False, 43, 37)(pp, -pp, send_p, rec_p)
    pd2 = pd.reshape(e_pad // 8, d)
    msgs = _edge_mlp(h1, pd2, w1c, w2T, b2r)
    partials = _agg_kernel(e_pad, n_sh, d)(rec_agg, msgs)
    update = _node_mlp(x, partials[0, :n], partials[1, :n],
                       w3aT, w3bT, b3r, w4T, b4r)
    return update


# speedup vs baseline: 2.1110x; 1.3944x over previous
import functools

import jax
import jax.numpy as jnp
from jax import lax
from jax.experimental import pallas as pl
from jax.experimental.pallas import tpu as pltpu
from jax.experimental.pallas import tpu_sc as plsc

F32 = jnp.float32

NC = 2
NS = 16
NW = NC * NS
CH = 128


def _round_up(a, m):
    return (a + m - 1) // m * m


def _tables_body(x_ref, wa_ref, wb_ref, b1_ref, ta_ref, tb_ref):
    xb = x_ref[...]
    ta_ref[...] = jnp.dot(xb, wa_ref[...], preferred_element_type=F32)
    tb_ref[...] = jnp.dot(xb, wb_ref[...],
                          preferred_element_type=F32) + b1_ref[...]


def _build_tables(x, waT, wbT, b1):
    n, d = x.shape
    bn = 1000
    grid = n // bn
    return pl.pallas_call(
        _tables_body,
        grid=(grid,),
        in_specs=[
            pl.BlockSpec((bn, d), lambda i: (i, 0)),
            pl.BlockSpec((d, d), lambda i: (0, 0)),
            pl.BlockSpec((d, d), lambda i: (0, 0)),
            pl.BlockSpec((1, d), lambda i: (0, 0)),
        ],
        out_specs=[
            pl.BlockSpec((bn, d), lambda i: (i, 0)),
            pl.BlockSpec((bn, d), lambda i: (i, 0)),
        ],
        out_shape=[
            jax.ShapeDtypeStruct((n, d), F32),
            jax.ShapeDtypeStruct((n, d), F32),
        ],
    )(x, waT, wbT, b1)


def _gather_kernel(e_pad, dw, tiled, p0, p1):
    assert NS * (p0 + p1) * 2 * CH == e_pad
    nslice = dw // 16
    mesh = plsc.VectorSubcoreMesh(
        core_axis_name="c", subcore_axis_name="s",
        num_cores=NC, num_subcores=NS)
    params = None if tiled else pltpu.CompilerParams(use_tc_tiling_on_sc=False)

    @functools.partial(
        pl.kernel,
        out_type=jax.ShapeDtypeStruct((e_pad, dw), F32),
        mesh=mesh,
        compiler_params=params,
        scratch_types=[
            pltpu.VMEM((CH,), jnp.int32),
            pltpu.VMEM((CH,), jnp.int32),
            pltpu.VMEM((CH, dw), F32),
            pltpu.VMEM((CH, dw), F32),
            pltpu.VMEM((CH,), jnp.int32),
            pltpu.VMEM((CH,), jnp.int32),
            pltpu.VMEM((CH, dw), F32),
            pltpu.VMEM((CH, dw), F32),
            pltpu.SemaphoreType.DMA,
            pltpu.SemaphoreType.DMA,
            pltpu.SemaphoreType.DMA,
            pltpu.SemaphoreType.DMA,
        ],
    )
    def k(ta_hbm, tb_hbm, send_hbm, rec_hbm, h_hbm,
          sidx0, ridx0, bs0, br0, sidx1, ridx1, bs1, br1, g0, g1, s0, s1):
        cid = lax.axis_index("c")
        sid = lax.axis_index("s")
        npair = jnp.where(cid == 0, p0, p1)
        base = jnp.where(cid == 0, sid * p0, NS * p0 + sid * p1) * 2 * CH

        def addstore(bs, br, off, sem):
            def addrow(r, c2):
                for j in range(nslice):
                    sl = pl.ds(j * 16, 16)
                    bs[r, sl] = bs[r, sl] + br[r, sl]
                return c2

            lax.fori_loop(0, CH, addrow, 0)
            pltpu.async_copy(bs, h_hbm.at[pl.ds(off, CH)], sem)

        def pair(j, carry):
            off0 = base + (2 * j) * CH
            off1 = off0 + CH

            @pl.when(j > 0)
            def _drain():
                pltpu.make_async_copy(
                    bs0, h_hbm.at[pl.ds(base, CH)], s0).wait()
                pltpu.make_async_copy(
                    bs1, h_hbm.at[pl.ds(base, CH)], s1).wait()

            pltpu.sync_copy(send_hbm.at[pl.ds(off0, CH)], sidx0)
            pltpu.sync_copy(rec_hbm.at[pl.ds(off0, CH)], ridx0)
            pltpu.async_copy(ta_hbm.at[sidx0], bs0, g0)
            pltpu.async_copy(tb_hbm.at[ridx0], br0, g0)
            pltpu.sync_copy(send_hbm.at[pl.ds(off1, CH)], sidx1)
            pltpu.sync_copy(rec_hbm.at[pl.ds(off1, CH)], ridx1)
            pltpu.async_copy(ta_hbm.at[sidx1], bs1, g1)
            pltpu.async_copy(tb_hbm.at[ridx1], br1, g1)

            pltpu.make_async_copy(ta_hbm.at[sidx0], bs0, g0).wait()
            pltpu.make_async_copy(tb_hbm.at[ridx0], br0, g0).wait()
            addstore(bs0, br0, off0, s0)
            pltpu.make_async_copy(ta_hbm.at[sidx1], bs1, g1).wait()
            pltpu.make_async_copy(tb_hbm.at[ridx1], br1, g1).wait()
            addstore(bs1, br1, off1, s1)
            return carry

        lax.fori_loop(0, npair, pair, 0)
        pltpu.make_async_copy(bs0, h_hbm.at[pl.ds(base, CH)], s0).wait()
        pltpu.make_async_copy(bs1, h_hbm.at[pl.ds(base, CH)], s1).wait()

    return k


def _edge_body(h_ref, pd_ref, w1c_ref, w2_ref, b2_ref, m_ref):
    d = w2_ref.shape[0]
    be = h_ref.shape[0]
    h1 = h_ref[...]
    pd = pd_ref[...]
    dz = pd.reshape(be // 8, 8, 16)
    d2 = jnp.sum(dz * dz, axis=2)
    dist = jnp.sqrt(d2 + 1e-12).reshape(be, 1)
    h = jax.nn.silu(h1 + dist * w1c_ref[...])
    t = jnp.dot(h, w2_ref[...], preferred_element_type=F32) + b2_ref[...]
    m_ref[...] = jax.nn.silu(t)


def _edge_mlp(h, pd2, w1c, w2T, b2):
    e_pad, d = h.shape
    be = 1024
    grid = e_pad // be
    return pl.pallas_call(
        _edge_body,
        grid=(grid,),
        in_specs=[
            pl.BlockSpec((be, d), lambda i: (i, 0)),
            pl.BlockSpec((be // 8, d), lambda i: (i, 0)),
            pl.BlockSpec((1, d), lambda i: (0, 0)),
            pl.BlockSpec((d, d), lambda i: (0, 0)),
            pl.BlockSpec((1, d), lambda i: (0, 0)),
        ],
        out_specs=pl.BlockSpec((be, d), lambda i: (i, 0)),
        out_shape=jax.ShapeDtypeStruct((e_pad, d), F32),
    )(h, pd2, w1c, w2T, b2)


def _agg_kernel(e_pad, n_sh, d):
    epw = e_pad // NW
    nchunk = epw // CH
    rows_per_tile = n_sh // NS
    ozchunk = rows_per_tile // CH
    mesh = plsc.VectorSubcoreMesh(
        core_axis_name="c", subcore_axis_name="s",
        num_cores=NC, num_subcores=NS)

    @functools.partial(
        pl.kernel,
        out_type=jax.ShapeDtypeStruct((NC, n_sh, d), F32),
        mesh=mesh,
        scratch_types=[
            pltpu.VMEM((CH,), jnp.int32),
            pltpu.VMEM((CH, d), F32),
            pltpu.VMEM_SHARED((n_sh, d), F32),
        ],
    )
    def k(rec_hbm, m_hbm, out_hbm, ridx, mbuf, shared):
        cid = lax.axis_index("c")
        sid = lax.axis_index("s")
        wid = sid * NC + cid
        tbase = sid * rows_per_tile

        def zrow(r, c2):
            for j in range(d // 16):
                mbuf[r, pl.ds(j * 16, 16)] = jnp.zeros((16,), F32)
            return c2

        lax.fori_loop(0, CH, zrow, 0)

        def zchunk(i, c2):
            pltpu.sync_copy(mbuf, shared.at[pl.ds(tbase + i * CH, CH)])
            return c2

        lax.fori_loop(0, ozchunk, zchunk, 0)
        plsc.subcore_barrier()

        base = wid * epw

        def chunk(i, c2):
            off = base + i * CH
            pltpu.sync_copy(rec_hbm.at[pl.ds(off, CH)], ridx)
            pltpu.sync_copy(m_hbm.at[pl.ds(off, CH)], mbuf)
            pltpu.sync_copy(mbuf, shared.at[ridx], add=True)
            return c2

        lax.fori_loop(0, nchunk, chunk, 0)
        plsc.subcore_barrier()

        def ochunk(i, c2):
            sl = pl.ds(tbase + i * CH, CH)
            pltpu.sync_copy(shared.at[sl], out_hbm.at[cid, sl])
            return c2

        lax.fori_loop(0, ozchunk, ochunk, 0)

    return k


def _node_body(x_ref, p0_ref, p1_ref, p2_ref, p3_ref,
               w3a_ref, w3b_ref, b3_ref, w4_ref, b4_ref, out_ref):
    xb = x_ref[...]
    aggr = (p0_ref[...] + p1_ref[...]) + (p2_ref[...] + p3_ref[...])
    u = jax.nn.silu(
        jnp.dot(xb, w3a_ref[...], preferred_element_type=F32)
        + jnp.dot(aggr, w3b_ref[...], preferred_element_type=F32)
        + b3_ref[...])
    out_ref[...] = jnp.dot(u, w4_ref[...], preferred_element_type=F32) \
        + b4_ref[...]


def _node_mlp(x, ps, w3aT, w3bT, b3, w4T, b4):
    n, d = x.shape
    bn = 1000
    grid = n // bn
    return pl.pallas_call(
        _node_body,
        grid=(grid,),
        in_specs=[pl.BlockSpec((bn, d), lambda i: (i, 0))] * 5 + [
            pl.BlockSpec((d, d), lambda i: (0, 0)),
            pl.BlockSpec((d, d), lambda i: (0, 0)),
            pl.BlockSpec((1, d), lambda i: (0, 0)),
            pl.BlockSpec((d, d), lambda i: (0, 0)),
            pl.BlockSpec((1, d), lambda i: (0, 0)),
        ],
        out_specs=pl.BlockSpec((bn, d), lambda i: (i, 0)),
        out_shape=jax.ShapeDtypeStruct((n, d), F32),
    )(x, *ps, w3aT, w3bT, b3, w4T, b4)


def kernel(x, pos, edge_index, W1, b1, W2, b2, W3, b3, W4, b4):
    n, d = x.shape
    e = edge_index.shape[1]
    nhalf = 2
    e_pad = _round_up(e, NW * CH * 2 * nhalf)
    e_half = e_pad // nhalf
    n_sh = _round_up(n + 1, NS * CH)

    waT = W1[:, :d].T
    wbT = W1[:, d:2 * d].T
    w1c = W1[:, 2 * d].reshape(1, d)
    b1r = b1.reshape(1, d)
    w2T = W2.T
    b2r = b2.reshape(1, d)
    w3aT = W3[:, :d].T
    w3bT = W3[:, d:].T
    b3r = b3.reshape(1, d)
    w4T = W4.T
    b4r = b4.reshape(1, d)

    send = edge_index[0]
    rec = edge_index[1]
    pad = e_pad - e
    spread = (jnp.arange(pad, dtype=jnp.int32) * 37) % n
    send_p = jnp.concatenate([send, spread])
    rec_p = jnp.concatenate([rec, spread])
    rec_agg = jnp.concatenate([rec, jnp.full((pad,), n, jnp.int32)])

    pp = jnp.concatenate([pos, jnp.zeros((n, 13), F32)], axis=1)

    ta, tb = _build_tables(x, waT, wbT, b1r)
    pm = -pp
    pb = e_half // (NS * 2 * CH) // 2
    gk = _gather_kernel(e_half, d, True, pb, pb)
    pk = _gather_kernel(e_half, 16, False, pb, pb)
    ak = _agg_kernel(e_half, n_sh, d)

    parts = []
    for k in range(nhalf):
        sl = slice(k * e_half, (k + 1) * e_half)
        h1 = gk(ta, tb, send_p[sl], rec_p[sl])
        pd = pk(pp, pm, send_p[sl], rec_p[sl])
        msgs = _edge_mlp(h1, pd.reshape(e_half // 8, d), w1c, w2T, b2r)
        parts.append(ak(rec_agg[sl], msgs))

    ps = [parts[k][c, :n] for k in range(nhalf) for c in range(NC)]
    update = _node_mlp(x, ps, w3aT, w3bT, b3r, w4T, b4r)
    return update


# speedup vs baseline: 2.2908x; 1.0852x over previous
import functools

import jax
import jax.numpy as jnp
from jax import lax
from jax.experimental import pallas as pl
from jax.experimental.pallas import tpu as pltpu
from jax.experimental.pallas import tpu_sc as plsc

F32 = jnp.float32

NC = 2
NS = 16
NW = NC * NS
CH = 128


def _round_up(a, m):
    return (a + m - 1) // m * m


def _tables_body(x_ref, wa_ref, wb_ref, b1_ref, ta_ref, tb_ref):
    xb = x_ref[...]
    ta_ref[...] = jnp.dot(xb, wa_ref[...], preferred_element_type=F32)
    tb_ref[...] = jnp.dot(xb, wb_ref[...],
                          preferred_element_type=F32) + b1_ref[...]


def _build_tables(x, waT, wbT, b1):
    n, d = x.shape
    bn = 1000
    grid = n // bn
    return pl.pallas_call(
        _tables_body,
        grid=(grid,),
        in_specs=[
            pl.BlockSpec((bn, d), lambda i: (i, 0)),
            pl.BlockSpec((d, d), lambda i: (0, 0)),
            pl.BlockSpec((d, d), lambda i: (0, 0)),
            pl.BlockSpec((1, d), lambda i: (0, 0)),
        ],
        out_specs=[
            pl.BlockSpec((bn, d), lambda i: (i, 0)),
            pl.BlockSpec((bn, d), lambda i: (i, 0)),
        ],
        out_shape=[
            jax.ShapeDtypeStruct((n, d), F32),
            jax.ShapeDtypeStruct((n, d), F32),
        ],
    )(x, waT, wbT, b1)


def _gather_kernel(e_pad, dw, tiled, p0, p1):
    assert NS * (p0 + p1) * 2 * CH == e_pad
    nslice = dw // 16
    mesh = plsc.VectorSubcoreMesh(
        core_axis_name="c", subcore_axis_name="s",
        num_cores=NC, num_subcores=NS)
    params = None if tiled else pltpu.CompilerParams(use_tc_tiling_on_sc=False)

    @functools.partial(
        pl.kernel,
        out_type=jax.ShapeDtypeStruct((e_pad, dw), F32),
        mesh=mesh,
        compiler_params=params,
        scratch_types=[
            pltpu.VMEM((CH,), jnp.int32),
            pltpu.VMEM((CH,), jnp.int32),
            pltpu.VMEM((CH, dw), F32),
            pltpu.VMEM((CH, dw), F32),
            pltpu.VMEM((CH,), jnp.int32),
            pltpu.VMEM((CH,), jnp.int32),
            pltpu.VMEM((CH, dw), F32),
            pltpu.VMEM((CH, dw), F32),
            pltpu.SemaphoreType.DMA,
            pltpu.SemaphoreType.DMA,
            pltpu.SemaphoreType.DMA,
            pltpu.SemaphoreType.DMA,
        ],
    )
    def k(ta_hbm, tb_hbm, send_hbm, rec_hbm, h_hbm,
          sidx0, ridx0, bs0, br0, sidx1, ridx1, bs1, br1, g0, g1, s0, s1):
        cid = lax.axis_index("c")
        sid = lax.axis_index("s")
        npair = jnp.where(cid == 0, p0, p1)
        base = jnp.where(cid == 0, sid * p0, NS * p0 + sid * p1) * 2 * CH

        def addstore(bs, br, off, sem):
            def addrow(r, c2):
                for j in range(nslice):
                    sl = pl.ds(j * 16, 16)
                    bs[r, sl] = bs[r, sl] + br[r, sl]
                return c2

            lax.fori_loop(0, CH, addrow, 0)
            pltpu.async_copy(bs, h_hbm.at[pl.ds(off, CH)], sem)

        def pair(j, carry):
            off0 = base + (2 * j) * CH
            off1 = off0 + CH

            @pl.when(j > 0)
            def _drain():
                pltpu.make_async_copy(
                    bs0, h_hbm.at[pl.ds(base, CH)], s0).wait()
                pltpu.make_async_copy(
                    bs1, h_hbm.at[pl.ds(base, CH)], s1).wait()

            pltpu.sync_copy(send_hbm.at[pl.ds(off0, CH)], sidx0)
            pltpu.sync_copy(rec_hbm.at[pl.ds(off0, CH)], ridx0)
            pltpu.async_copy(ta_hbm.at[sidx0], bs0, g0)
            pltpu.async_copy(tb_hbm.at[ridx0], br0, g0)
            pltpu.sync_copy(send_hbm.at[pl.ds(off1, CH)], sidx1)
            pltpu.sync_copy(rec_hbm.at[pl.ds(off1, CH)], ridx1)
            pltpu.async_copy(ta_hbm.at[sidx1], bs1, g1)
            pltpu.async_copy(tb_hbm.at[ridx1], br1, g1)

            pltpu.make_async_copy(ta_hbm.at[sidx0], bs0, g0).wait()
            pltpu.make_async_copy(tb_hbm.at[ridx0], br0, g0).wait()
            addstore(bs0, br0, off0, s0)
            pltpu.make_async_copy(ta_hbm.at[sidx1], bs1, g1).wait()
            pltpu.make_async_copy(tb_hbm.at[ridx1], br1, g1).wait()
            addstore(bs1, br1, off1, s1)
            return carry

        lax.fori_loop(0, npair, pair, 0)
        pltpu.make_async_copy(bs0, h_hbm.at[pl.ds(base, CH)], s0).wait()
        pltpu.make_async_copy(bs1, h_hbm.at[pl.ds(base, CH)], s1).wait()

    return k


def _edge_body(h_ref, pd_ref, w1c_ref, w2_ref, b2_ref, m_ref):
    d = w2_ref.shape[0]
    be = h_ref.shape[0]
    h1 = h_ref[...]
    pd = pd_ref[...]
    dz = pd.reshape(be // 8, 8, 16)
    d2 = jnp.sum(dz * dz, axis=2)
    dist = jnp.sqrt(d2 + 1e-12).reshape(be, 1)
    h = jax.nn.silu(h1 + dist * w1c_ref[...])
    t = jnp.dot(h, w2_ref[...], preferred_element_type=F32) + b2_ref[...]
    m_ref[...] = jax.nn.silu(t)


def _edge_mlp(h, pd2, w1c, w2T, b2):
    e_pad, d = h.shape
    be = 2048
    grid = e_pad // be
    return pl.pallas_call(
        _edge_body,
        grid=(grid,),
        in_specs=[
            pl.BlockSpec((be, d), lambda i: (i, 0)),
            pl.BlockSpec((be // 8, d), lambda i: (i, 0)),
            pl.BlockSpec((1, d), lambda i: (0, 0)),
            pl.BlockSpec((d, d), lambda i: (0, 0)),
            pl.BlockSpec((1, d), lambda i: (0, 0)),
        ],
        out_specs=pl.BlockSpec((be, d), lambda i: (i, 0)),
        out_shape=jax.ShapeDtypeStruct((e_pad, d), F32),
    )(h, pd2, w1c, w2T, b2)


def _agg_kernel(e_pad, n_sh, d):
    epw = e_pad // NW
    nchunk = epw // CH
    rows_per_tile = n_sh // NS
    ozchunk = rows_per_tile // CH
    mesh = plsc.VectorSubcoreMesh(
        core_axis_name="c", subcore_axis_name="s",
        num_cores=NC, num_subcores=NS)

    npair = nchunk // 2

    @functools.partial(
        pl.kernel,
        out_type=jax.ShapeDtypeStruct((NC, n_sh, d), F32),
        mesh=mesh,
        scratch_types=[
            pltpu.VMEM((CH,), jnp.int32),
            pltpu.VMEM((CH, d), F32),
            pltpu.VMEM((CH,), jnp.int32),
            pltpu.VMEM((CH, d), F32),
            pltpu.VMEM_SHARED((n_sh, d), F32),
            pltpu.SemaphoreType.DMA,
            pltpu.SemaphoreType.DMA,
            pltpu.SemaphoreType.DMA,
            pltpu.SemaphoreType.DMA,
        ],
    )
    def k(rec_hbm, m_hbm, out_hbm, ridx0, mb0, ridx1, mb1, shared,
          g0, g1, s0, s1):
        cid = lax.axis_index("c")
        sid = lax.axis_index("s")
        wid = sid * NC + cid
        tbase = sid * rows_per_tile

        def zrow(r, c2):
            for j in range(d // 16):
                mb0[r, pl.ds(j * 16, 16)] = jnp.zeros((16,), F32)
            return c2

        lax.fori_loop(0, CH, zrow, 0)

        def zchunk(i, c2):
            pltpu.sync_copy(mb0, shared.at[pl.ds(tbase + i * CH, CH)])
            return c2

        lax.fori_loop(0, ozchunk, zchunk, 0)
        plsc.subcore_barrier()

        base = wid * epw

        def pair(j, c2):
            off0 = base + (2 * j) * CH
            off1 = off0 + CH

            @pl.when(j > 0)
            def _drain():
                pltpu.make_async_copy(mb0, shared.at[ridx0], s0).wait()
                pltpu.make_async_copy(mb1, shared.at[ridx1], s1).wait()

            pltpu.sync_copy(rec_hbm.at[pl.ds(off0, CH)], ridx0)
            pltpu.async_copy(m_hbm.at[pl.ds(off0, CH)], mb0, g0)
            pltpu.sync_copy(rec_hbm.at[pl.ds(off1, CH)], ridx1)
            pltpu.async_copy(m_hbm.at[pl.ds(off1, CH)], mb1, g1)
            pltpu.make_async_copy(m_hbm.at[pl.ds(off0, CH)], mb0, g0).wait()
            pltpu.async_copy(mb0, shared.at[ridx0], s0, add=True)
            pltpu.make_async_copy(m_hbm.at[pl.ds(off1, CH)], mb1, g1).wait()
            pltpu.async_copy(mb1, shared.at[ridx1], s1, add=True)
            return c2

        lax.fori_loop(0, npair, pair, 0)
        pltpu.make_async_copy(mb0, shared.at[ridx0], s0).wait()
        pltpu.make_async_copy(mb1, shared.at[ridx1], s1).wait()
        plsc.subcore_barrier()

        def ochunk(i, c2):
            sl = pl.ds(tbase + i * CH, CH)
            pltpu.sync_copy(shared.at[sl], out_hbm.at[cid, sl])
            return c2

        lax.fori_loop(0, ozchunk, ochunk, 0)

    return k


def _node_body(x_ref, p0_ref, p1_ref, p2_ref, p3_ref,
               w3a_ref, w3b_ref, b3_ref, w4_ref, b4_ref, out_ref):
    xb = x_ref[...]
    aggr = (p0_ref[...] + p1_ref[...]) + (p2_ref[...] + p3_ref[...])
    u = jax.nn.silu(
        jnp.dot(xb, w3a_ref[...], preferred_element_type=F32)
        + jnp.dot(aggr, w3b_ref[...], preferred_element_type=F32)
        + b3_ref[...])
    out_ref[...] = jnp.dot(u, w4_ref[...], preferred_element_type=F32) \
        + b4_ref[...]


def _node_mlp(x, ps, w3aT, w3bT, b3, w4T, b4):
    n, d = x.shape
    bn = 1000
    grid = n // bn
    return pl.pallas_call(
        _node_body,
        grid=(grid,),
        in_specs=[pl.BlockSpec((bn, d), lambda i: (i, 0))] * 5 + [
            pl.BlockSpec((d, d), lambda i: (0, 0)),
            pl.BlockSpec((d, d), lambda i: (0, 0)),
            pl.BlockSpec((1, d), lambda i: (0, 0)),
            pl.BlockSpec((d, d), lambda i: (0, 0)),
            pl.BlockSpec((1, d), lambda i: (0, 0)),
        ],
        out_specs=pl.BlockSpec((bn, d), lambda i: (i, 0)),
        out_shape=jax.ShapeDtypeStruct((n, d), F32),
    )(x, *ps, w3aT, w3bT, b3, w4T, b4)


def kernel(x, pos, edge_index, W1, b1, W2, b2, W3, b3, W4, b4):
    n, d = x.shape
    e = edge_index.shape[1]
    nhalf = 2
    e_pad = _round_up(e, NW * CH * 2 * nhalf)
    e_half = e_pad // nhalf
    n_sh = _round_up(n + 1, NS * CH)

    waT = W1[:, :d].T
    wbT = W1[:, d:2 * d].T
    w1c = W1[:, 2 * d].reshape(1, d)
    b1r = b1.reshape(1, d)
    w2T = W2.T
    b2r = b2.reshape(1, d)
    w3aT = W3[:, :d].T
    w3bT = W3[:, d:].T
    b3r = b3.reshape(1, d)
    w4T = W4.T
    b4r = b4.reshape(1, d)

    send = edge_index[0]
    rec = edge_index[1]
    pad = e_pad - e
    spread = (jnp.arange(pad, dtype=jnp.int32) * 37) % n
    send_p = jnp.concatenate([send, spread])
    rec_p = jnp.concatenate([rec, spread])
    rec_agg = jnp.concatenate([rec, jnp.full((pad,), n, jnp.int32)])

    pp = jnp.concatenate([pos, jnp.zeros((n, 13), F32)], axis=1)

    ta, tb = _build_tables(x, waT, wbT, b1r)
    pm = -pp
    pb = e_half // (NS * 2 * CH) // 2
    gk = _gather_kernel(e_half, d, True, pb, pb)
    pk = _gather_kernel(e_half, 16, False, pb, pb)
    ak = _agg_kernel(e_half, n_sh, d)

    parts = []
    for k in range(nhalf):
        sl = slice(k * e_half, (k + 1) * e_half)
        h1 = gk(ta, tb, send_p[sl], rec_p[sl])
        pd = pk(pp, pm, send_p[sl], rec_p[sl])
        msgs = _edge_mlp(h1, pd.reshape(e_half // 8, d), w1c, w2T, b2r)
        parts.append(ak(rec_agg[sl], msgs))

    ps = [parts[k][c, :n] for k in range(nhalf) for c in range(NC)]
    update = _node_mlp(x, ps, w3aT, w3bT, b3r, w4T, b4r)
    return update
